# baseline (device time: 51039 ns/iter reference)
import functools

import jax
import jax.numpy as jnp
from jax import lax
from jax.experimental import pallas as pl
from jax.experimental.pallas import tpu as pltpu

N_DEV = 4

ORDER = (2, 1, 3, 0)


def kernel(x, w_mat):
    m_per, k_dim = x.shape
    n_total = w_mat.shape[1]
    n_per = n_total // N_DEV
    m_total = N_DEV * m_per

    def body(x_hbm, w_hbm, out_ref, x_v, w_v, send_buf, recv_buf,
             x_sem, w_sems, send_sems, recv_sems):
        my = lax.axis_index("i")

        x_copy = pltpu.make_async_copy(x_hbm, x_v, x_sem)
        x_copy.start()

        def w_copy(idx, buf):
            tgt = (my + ORDER[idx]) % N_DEV
            return pltpu.make_async_copy(
                w_hbm.at[:, pl.ds(tgt * n_per, n_per)],
                w_v.at[buf],
                w_sems.at[buf],
            )

        w_copies = [w_copy(0, 0)]
        w_copies[0].start()

        barrier = pltpu.get_barrier_semaphore()
        for k in range(1, N_DEV):
            peer = (my + k) % N_DEV
            pl.semaphore_signal(
                barrier, inc=1,
                device_id=(peer,), device_id_type=pl.DeviceIdType.MESH,
            )
        pl.semaphore_wait(barrier, N_DEV - 1)

        x_copy.wait()

        sends = []
        for idx, k in enumerate(ORDER):
            tgt = (my + k) % N_DEV
            cur = idx % 2
            if idx + 1 < N_DEV:
                nxt = w_copy(idx + 1, (idx + 1) % 2)
                nxt.start()
                w_copies.append(nxt)
            w_copies[idx].wait()
            block = jnp.dot(
                x_v[:, :], w_v[cur, :, :],
                preferred_element_type=jnp.float32,
            )
            if k == 0:
                out_ref[pl.ds(my * m_per, m_per), :] = block
            else:
                slot = idx
                send_buf[slot, :, :] = block.astype(jnp.bfloat16)
                rdma = pltpu.make_async_remote_copy(
                    src_ref=send_buf.at[slot],
                    dst_ref=recv_buf.at[my],
                    send_sem=send_sems.at[slot],
                    recv_sem=recv_sems.at[my],
                    device_id=(tgt,),
                    device_id_type=pl.DeviceIdType.MESH,
                )
                rdma.start()
                sends.append(rdma)

        for k in ORDER[:-1]:
            src = (my + k) % N_DEV
            recv = pltpu.make_async_remote_copy(
                src_ref=send_buf.at[0],
                dst_ref=recv_buf.at[src],
                send_sem=send_sems.at[0],
                recv_sem=recv_sems.at[src],
                device_id=(src,),
                device_id_type=pl.DeviceIdType.MESH,
            )
            recv.wait_recv()
            out_ref[pl.ds(src * m_per, m_per), :] = recv_buf[
                src, :, :
            ].astype(jnp.float32)
        for rdma in sends:
            rdma.wait_send()

        @functools.partial(
            pl.run_scoped, exit_sem=pltpu.SemaphoreType.REGULAR
        )
        def _(exit_sem):
            for k in range(1, N_DEV):
                peer = (my + k) % N_DEV
                pl.semaphore_signal(
                    exit_sem, inc=1,
                    device_id=(peer,), device_id_type=pl.DeviceIdType.MESH,
                )
            pl.semaphore_wait(exit_sem, N_DEV - 1)

    return pl.pallas_call(
        body,
        out_shape=jax.ShapeDtypeStruct((m_total, n_per), jnp.float32),
        in_specs=[
            pl.BlockSpec(memory_space=pltpu.MemorySpace.HBM),
            pl.BlockSpec(memory_space=pltpu.MemorySpace.HBM),
        ],
        out_specs=pl.BlockSpec(memory_space=pltpu.VMEM),
        scratch_shapes=[
            pltpu.VMEM((m_per, k_dim), jnp.float32),
            pltpu.VMEM((2, k_dim, n_per), jnp.float32),
            pltpu.VMEM((N_DEV, m_per, n_per), jnp.bfloat16),
            pltpu.VMEM((N_DEV, m_per, n_per), jnp.bfloat16),
            pltpu.SemaphoreType.DMA,
            pltpu.SemaphoreType.DMA((2,)),
            pltpu.SemaphoreType.DMA((N_DEV,)),
            pltpu.SemaphoreType.DMA((N_DEV,)),
        ],
        compiler_params=pltpu.CompilerParams(
            collective_id=0,
            vmem_limit_bytes=100 * 1024 * 1024,
        ),
    )(x, w_mat)


# device time: 49099 ns/iter; 1.0395x vs baseline; 1.0395x over previous
import functools

import jax
import jax.numpy as jnp
from jax import lax
from jax.experimental import pallas as pl
from jax.experimental.pallas import tpu as pltpu

N_DEV = 4
N_HALF = 2

ORDER = (2, 1, 3, 0)


def kernel(x, w_mat):
    m_per, k_dim = x.shape
    n_total = w_mat.shape[1]
    n_per = n_total // N_DEV
    m_total = N_DEV * m_per
    m_half = m_per // N_HALF

    def body(x_hbm, w_hbm, out_ref, x_v, w_v, send_buf, recv_buf,
             x_sems, w_sems, send_sems, recv_sems):
        my = lax.axis_index("i")

        x_copies = []
        for h in range(N_HALF):
            c = pltpu.make_async_copy(
                x_hbm.at[pl.ds(h * m_half, m_half), :],
                x_v.at[pl.ds(h * m_half, m_half), :],
                x_sems.at[h],
            )
            c.start()
            x_copies.append(c)

        w_copies = []
        for idx, k in enumerate(ORDER):
            tgt = (my + k) % N_DEV
            c = pltpu.make_async_copy(
                w_hbm.at[:, pl.ds(tgt * n_per, n_per)],
                w_v.at[idx],
                w_sems.at[idx],
            )
            c.start()
            w_copies.append(c)

        barrier = pltpu.get_barrier_semaphore()
        for k in range(1, N_DEV):
            peer = (my + k) % N_DEV
            pl.semaphore_signal(
                barrier, inc=1,
                device_id=(peer,), device_id_type=pl.DeviceIdType.MESH,
            )
        pl.semaphore_wait(barrier, N_DEV - 1)

        sends = []
        for h in range(N_HALF):
            x_copies[h].wait()
            for idx, k in enumerate(ORDER):
                tgt = (my + k) % N_DEV
                if h == 0:
                    w_copies[idx].wait()
                block = jnp.dot(
                    x_v[pl.ds(h * m_half, m_half), :], w_v[idx, :, :],
                    preferred_element_type=jnp.float32,
                )
                if k == 0:
                    out_ref[pl.ds(my * m_per + h * m_half, m_half), :] = block
                else:
                    slot = idx * N_HALF + h
                    send_buf[slot, :, :] = block.astype(jnp.bfloat16)
                    rdma = pltpu.make_async_remote_copy(
                        src_ref=send_buf.at[slot],
                        dst_ref=recv_buf.at[my, pl.ds(h * m_half, m_half), :],
                        send_sem=send_sems.at[slot],
                        recv_sem=recv_sems.at[my, h],
                        device_id=(tgt,),
                        device_id_type=pl.DeviceIdType.MESH,
                    )
                    rdma.start()
                    sends.append(rdma)

        for h in range(N_HALF):
            for k in ORDER[:-1]:
                src = (my + k) % N_DEV
                recv = pltpu.make_async_remote_copy(
                    src_ref=send_buf.at[0],
                    dst_ref=recv_buf.at[src, pl.ds(h * m_half, m_half), :],
                    send_sem=send_sems.at[0],
                    recv_sem=recv_sems.at[src, h],
                    device_id=(src,),
                    device_id_type=pl.DeviceIdType.MESH,
                )
                recv.wait_recv()
                out_ref[pl.ds(src * m_per + h * m_half, m_half), :] = (
                    recv_buf[src, pl.ds(h * m_half, m_half), :]
                    .astype(jnp.float32)
                )
        for rdma in sends:
            rdma.wait_send()

        @functools.partial(
            pl.run_scoped, exit_sem=pltpu.SemaphoreType.REGULAR
        )
        def _(exit_sem):
            for k in range(1, N_DEV):
                peer = (my + k) % N_DEV
                pl.semaphore_signal(
                    exit_sem, inc=1,
                    device_id=(peer,), device_id_type=pl.DeviceIdType.MESH,
                )
            pl.semaphore_wait(exit_sem, N_DEV - 1)

    return pl.pallas_call(
        body,
        out_shape=jax.ShapeDtypeStruct((m_total, n_per), jnp.float32),
        in_specs=[
            pl.BlockSpec(memory_space=pltpu.MemorySpace.HBM),
            pl.BlockSpec(memory_space=pltpu.MemorySpace.HBM),
        ],
        out_specs=pl.BlockSpec(memory_space=pltpu.VMEM),
        scratch_shapes=[
            pltpu.VMEM((m_per, k_dim), jnp.float32),
            pltpu.VMEM((N_DEV, k_dim, n_per), jnp.float32),
            pltpu.VMEM(((N_DEV - 1) * N_HALF, m_half, n_per),
                       jnp.bfloat16),
            pltpu.VMEM((N_DEV, m_per, n_per), jnp.bfloat16),
            pltpu.SemaphoreType.DMA((N_HALF,)),
            pltpu.SemaphoreType.DMA((N_DEV,)),
            pltpu.SemaphoreType.DMA(((N_DEV - 1) * N_HALF,)),
            pltpu.SemaphoreType.DMA((N_DEV, N_HALF)),
        ],
        compiler_params=pltpu.CompilerParams(
            collective_id=0,
            vmem_limit_bytes=110 * 1024 * 1024,
        ),
    )(x, w_mat)
